# baseline (speedup 1.0000x reference)
import functools
import jax
import jax.numpy as jnp
from jax import lax
from jax.experimental import pallas as pl
from jax.experimental.pallas import tpu as pltpu
from jax.experimental.pallas import tpu_sc as plsc

D = 64
SCALE = 8.0
BB = 128
NS_UNITS = 200


def _build_b(NC: int, NS: int):
    mesh = plsc.VectorSubcoreMesh(core_axis_name="c", subcore_axis_name="s")

    @functools.partial(
        pl.kernel,
        mesh=mesh,
        out_type=jax.ShapeDtypeStruct((200, 8, 32, 8, 128), jnp.float32),
        scratch_types=[
            pltpu.VMEM((2, 1, BB), jnp.int32),
            pltpu.VMEM((2, BB, D), jnp.float32),
            pltpu.VMEM((2, 8, 8, 129), jnp.float32),
            pltpu.SemaphoreType.DMA,
            pltpu.SemaphoreType.DMA,
            pltpu.SemaphoreType.DMA,
        ],
        compiler_params=pltpu.CompilerParams(
            use_tc_tiling_on_sc=False, needs_layout_passes=False
        ),
    )
    def kb(idxt_hbm, table_hbm, out_hbm, idx_v, g_buf, t_buf, sem_i, sem_g, sem_o):
        cid = lax.axis_index("c")
        sid = lax.axis_index("s")
        w = sid * NC + cid
        col0 = pl.multiple_of(w * BB, BB)

        iota = lax.iota(jnp.int32, 16)
        c8_vec = lax.bitwise_and(iota, 7)
        ctb_vec = lax.shift_right_logical(iota, 3)

        def idx_src(s):
            return idxt_hbm.at[pl.ds(s, 1), pl.ds(col0, BB)]

        def fire_idx(s, b):
            pltpu.async_copy(idx_src(s), idx_v.at[b], sem_i)

        def drain_idx(s, b):
            pltpu.make_async_copy(idx_src(s), idx_v.at[b], sem_i).wait()

        def fire_gather(s, b):
            pltpu.async_copy(table_hbm.at[idx_v.at[b, 0]], g_buf.at[b], sem_g)

        def drain_gather(s, b):
            pltpu.make_async_copy(
                table_hbm.at[idx_v.at[b, 0]], g_buf.at[b], sem_g
            ).wait()

        def out_dst(s):
            return out_hbm.at[s, :, w]

        def fire_out(s, b):
            pltpu.async_copy(t_buf.at[b, :, :, pl.ds(0, 128)], out_dst(s), sem_o)

        def drain_out(s, b):
            pltpu.make_async_copy(
                t_buf.at[b, :, :, pl.ds(0, 128)], out_dst(s), sem_o
            ).wait()

        cta = [ctb_vec + 2 * g for g in range(D // 16)]
        zeros16 = jnp.zeros((16,), dtype=jnp.int32)

        def transpose_scale(b):
            tb = t_buf.at[b]
            gb = g_buf.at[b]

            @plsc.parallel_loop(0, BB, unroll=8, carry=zeros16)
            def row_body(r, bsp):
                for g in range(D // 16):
                    v = gb[r, pl.ds(g * 16, 16)] * SCALE
                    plsc.store_scatter(tb, [cta[g], c8_vec, bsp], v)
                return bsp + 1

        fire_idx(0, 0)
        drain_idx(0, 0)
        fire_gather(0, 0)
        fire_idx(1, 1)
        drain_idx(1, 1)
        drain_gather(0, 0)
        fire_gather(1, 1)
        fire_idx(2, 0)
        transpose_scale(0)
        fire_out(0, 0)

        def pair_body(i, carry):
            t = 1 + 2 * i
            for b in (1, 0):
                u = t if b == 1 else t + 1
                drain_idx(u + 1, 1 - b)
                drain_gather(u, b)
                fire_gather(u + 1, 1 - b)

                @pl.when(u < NS_UNITS - 2)
                def _():
                    fire_idx(u + 2, b)

                transpose_scale(b)
                drain_out(u - 1, 1 - b)
                fire_out(u, b)
            return carry

        lax.fori_loop(0, (NS_UNITS - 2) // 2, pair_body, 0)

        gl = NS_UNITS - 1
        drain_gather(gl, 1)
        transpose_scale(1)
        drain_out(gl - 1, 0)
        fire_out(gl, 1)
        drain_out(gl, 1)

    return kb


def kernel(pattern_ids, embedding_weight):
    S0, S1 = pattern_ids.shape
    idxt = pattern_ids.astype(jnp.int32).T
    info = plsc.get_sparse_core_info()
    kb = _build_b(info.num_cores, info.num_subcores)
    out5 = kb(idxt, embedding_weight)
    return out5.transpose(2, 4, 0, 1, 3).reshape(S0, S1, D)
